# baseline (speedup 1.0000x reference)
import functools

import jax
import jax.numpy as jnp
from jax import lax
from jax.experimental import pallas as pl
from jax.experimental.pallas import tpu as pltpu
from jax.experimental.pallas import tpu_sc as plsc

DIM = 64
HIST = 200
HIST_PAD = 208
HALF = HIST_PAD // 2
N_GROUPS = HIST_PAD // 16
N_VCHUNKS = (2 * DIM) // 16


def _body(idx_hbm, uidx_hbm, item_hbm, user_hbm, w1_hbm, b1_hbm, w2_hbm,
          b2_hbm, out_hbm, idx_v, uidx_v, rows_v, urow_v, w1_v, b1_v, w2_v,
          b2_v, out_v, sem1, sem2, sem3):
    is_worker = (lax.axis_index("c") == 0) & (lax.axis_index("s") == 0)

    @pl.when(is_worker)
    def _():
        pltpu.sync_copy(idx_hbm, idx_v)
        pltpu.sync_copy(uidx_hbm, uidx_v)
        g1 = pltpu.async_copy(item_hbm.at[idx_v.at[pl.ds(0, HALF)]],
                              rows_v.at[pl.ds(0, HALF)], sem1)
        g2 = pltpu.async_copy(item_hbm.at[idx_v.at[pl.ds(HALF, HALF)]],
                              rows_v.at[pl.ds(HALF, HALF)], sem2)
        g3 = pltpu.async_copy(user_hbm.at[uidx_v], urow_v, sem3)
        pltpu.sync_copy(w1_hbm, w1_v)
        pltpu.sync_copy(b1_hbm, b1_v)
        pltpu.sync_copy(w2_hbm, w2_v)
        pltpu.sync_copy(b2_hbm, b2_v)

        lanes = lax.iota(jnp.int32, 16)
        f32 = jnp.float32

        w2c = [w2_v[pl.ds(c * 16, 16)] for c in range(DIM // 16)]
        accs_v = [jnp.zeros((16,), f32) for _ in range(N_VCHUNKS)]
        for k in range(DIM):
            w2b = jnp.full((16,), w2c[k // 16][k % 16], f32)
            colk = jnp.full((16,), k, jnp.int32)
            for cidx in range(N_VCHUNKS):
                got = plsc.load_gather(w1_v, [lanes + cidx * 16, colk])
                accs_v[cidx] = accs_v[cidx] + got * w2b

        g3.wait()
        cvec = jnp.zeros((16,), f32)
        for cidx in range(4):
            sl = pl.ds(cidx * 16, 16)
            cvec = cvec + urow_v[0, sl] * accs_v[cidx]
            cvec = cvec + b1_v[sl] * w2c[cidx]
        c_const = jnp.sum(cvec) + b2_v[pl.ds(0, 16)][0]

        g1.wait()
        g2.wait()
        accs = [jnp.zeros((16,), f32) for _ in range(N_GROUPS)]
        for d in range(DIM):
            vb = jnp.full((16,), accs_v[4 + d // 16][d % 16], f32)
            cold = jnp.full((16,), d, jnp.int32)
            for g in range(N_GROUPS):
                got = plsc.load_gather(rows_v, [lanes + g * 16, cold])
                accs[g] = accs[g] + got * vb

        logits = []
        for g in range(N_GROUPS):
            l = accs[g] + c_const
            l = jnp.where(l >= 0.0, l, 0.01 * l)
            if (g + 1) * 16 > HIST:
                l = jnp.where(lanes + g * 16 < HIST, l, -1e30)
            logits.append(l)
        mvec = logits[0]
        for g in range(1, N_GROUPS):
            mvec = jnp.maximum(mvec, logits[g])
        m = jnp.max(mvec)
        exps = [jnp.exp(l - m) for l in logits]
        svec = exps[0]
        for g in range(1, N_GROUPS):
            svec = svec + exps[g]
        sb = jnp.full((16,), jnp.sum(svec), f32)
        inv = jnp.ones((16,), f32) / sb
        for g in range(N_GROUPS):
            out_v[pl.ds(g * 16, 16)] = exps[g] * inv

        pltpu.sync_copy(out_v.at[pl.ds(0, HIST)], out_hbm)


_sc_kernel = functools.partial(
    pl.kernel,
    out_type=jax.ShapeDtypeStruct((HIST,), jnp.float32),
    mesh=plsc.VectorSubcoreMesh(core_axis_name="c", subcore_axis_name="s"),
    compiler_params=pltpu.CompilerParams(needs_layout_passes=False,
                                         use_tc_tiling_on_sc=False),
    scratch_types=[
        pltpu.VMEM((HIST_PAD,), jnp.int32),
        pltpu.VMEM((8,), jnp.int32),
        pltpu.VMEM((HIST_PAD, DIM), jnp.float32),
        pltpu.VMEM((8, DIM), jnp.float32),
        pltpu.VMEM((2 * DIM, DIM), jnp.float32),
        pltpu.VMEM((DIM,), jnp.float32),
        pltpu.VMEM((DIM,), jnp.float32),
        pltpu.VMEM((16,), jnp.float32),
        pltpu.VMEM((HIST_PAD,), jnp.float32),
        pltpu.SemaphoreType.DMA,
        pltpu.SemaphoreType.DMA,
        pltpu.SemaphoreType.DMA,
    ],
)(_body)


def kernel(user_indice, interacted_item_indices, user_table, item_table,
           W1, b1, W2, b2):
    idx = jnp.concatenate([
        interacted_item_indices.astype(jnp.int32),
        jnp.zeros((HIST_PAD - HIST,), jnp.int32),
    ])
    uidx = jnp.full((8,), user_indice, dtype=jnp.int32)
    w2 = W2.reshape(DIM)
    b2p = jnp.concatenate([b2, jnp.zeros((15,), jnp.float32)])
    return _sc_kernel(idx, uidx, item_table, user_table, W1, b1, w2, b2p)
